# baseline (speedup 1.0000x reference)
import functools

import jax
import jax.numpy as jnp
from jax import lax
from jax.experimental import pallas as pl
from jax.experimental.pallas import tpu as pltpu
from jax.experimental.pallas import tpu_sc as plsc

B = 16
SEQ = 2048
TOTAL = B * SEQ
H = 1024
V = 100000
L = 16
HC = H // L

ROWS_PER_BLOCK = 2048

NB_SC = 8
TC_ROWS = (B - NB_SC) * SEQ
SC_ROWS = NB_SC * SEQ
SC_BASE_ROW = (B - NB_SC) * SEQ
NTILE = 32
ROWS_PER_TILE = SC_ROWS // NTILE
RCHUNK = 16
NCH = ROWS_PER_TILE // RCHUNK
NPAIR = NCH // 2


def _tw_body(hs_ref, w_ref, b_ref, out_ref):
    acc = jnp.dot(hs_ref[...], w_ref[...], preferred_element_type=jnp.float32)
    out_ref[...] = jnp.maximum(acc + b_ref[0, 0], 0.0)


def _token_weights_tc(hidden_states, W, b):
    return pl.pallas_call(
        _tw_body,
        grid=(TC_ROWS // ROWS_PER_BLOCK,),
        in_specs=[
            pl.BlockSpec((ROWS_PER_BLOCK, H), lambda i: (i, 0)),
            pl.BlockSpec((H, 1), lambda i: (0, 0)),
            pl.BlockSpec(memory_space=pltpu.SMEM),
        ],
        out_specs=pl.BlockSpec((ROWS_PER_BLOCK, 1), lambda i: (i, 0)),
        out_shape=jax.ShapeDtypeStruct((TC_ROWS, 1), jnp.float32),
    )(hidden_states, W, b.reshape(1, 1))


def _sc_matvec_body(hs_hbm, w_hbm, b_hbm, tw_hbm, buf0, buf1, w_v, b_v, tw_v,
                    sem0, sem1):
    cid = lax.axis_index("c")
    sid = lax.axis_index("s")
    tile = sid * 2 + cid
    row0 = SC_BASE_ROW + tile * ROWS_PER_TILE

    pltpu.sync_copy(w_hbm, w_v)
    pltpu.sync_copy(b_hbm, b_v)
    bias = b_v[...]
    lanes = lax.broadcasted_iota(jnp.int32, (L,), 0)

    def start(g, buf, sem):
        pltpu.async_copy(hs_hbm.at[pl.ds(row0 + g * RCHUNK, RCHUNK), :], buf, sem)

    def wait(g, buf, sem):
        pltpu.make_async_copy(
            hs_hbm.at[pl.ds(row0 + g * RCHUNK, RCHUNK), :], buf, sem
        ).wait()

    def compute(buf, g):
        def c_body(c, accs):
            wv = w_v[pl.ds(c * L, L)]
            return tuple(
                accs[r] + buf[r, pl.ds(c * L, L)] * wv for r in range(RCHUNK)
            )

        accs = lax.fori_loop(
            0, HC, c_body,
            tuple(jnp.zeros((L,), jnp.float32) for _ in range(RCHUNK)),
        )
        res = jnp.zeros((L,), jnp.float32)
        for r in range(RCHUNK):
            res = jnp.where(lanes == r, jnp.sum(accs[r]), res)
        tw_v[pl.ds(g * RCHUNK, RCHUNK)] = jnp.maximum(res + bias, 0.0)

    start(0, buf0, sem0)
    start(1, buf1, sem1)

    def pair_body(p, _):
        g0 = p * 2
        wait(g0, buf0, sem0)
        compute(buf0, g0)

        @pl.when(p < NPAIR - 1)
        def _():
            start(g0 + 2, buf0, sem0)

        g1 = g0 + 1
        wait(g1, buf1, sem1)
        compute(buf1, g1)

        @pl.when(p < NPAIR - 1)
        def _():
            start(g1 + 2, buf1, sem1)

        return ()

    lax.fori_loop(0, NPAIR, pair_body, ())

    pltpu.sync_copy(tw_v, tw_hbm.at[pl.ds(tile * ROWS_PER_TILE, ROWS_PER_TILE)])


_sc_matvec = functools.partial(
    pl.kernel,
    out_type=jax.ShapeDtypeStruct((SC_ROWS,), jnp.float32),
    mesh=plsc.VectorSubcoreMesh(core_axis_name="c", subcore_axis_name="s"),
    compiler_params=pltpu.CompilerParams(needs_layout_passes=False),
    scratch_types=[
        pltpu.VMEM((RCHUNK, H), jnp.float32),
        pltpu.VMEM((RCHUNK, H), jnp.float32),
        pltpu.VMEM((H,), jnp.float32),
        pltpu.VMEM((L,), jnp.float32),
        pltpu.VMEM((ROWS_PER_TILE,), jnp.float32),
        pltpu.SemaphoreType.DMA,
        pltpu.SemaphoreType.DMA,
    ],
)(_sc_matvec_body)


HALF_V = V // 2


def _sc_scatter_body(ids_hbm, tw_tc_hbm, tw_sc_hbm, out_hbm, table_v, ids_v, tw_v):
    cid = lax.axis_index("c")
    sid = lax.axis_index("s")
    wid = sid * 2 + cid
    batch = wid // 2
    lo = (wid % 2) * HALF_V

    zeros = jnp.zeros((L,), jnp.float32)

    def zero_body(j, _):
        table_v[pl.ds(j * L, L)] = zeros
        return ()

    lax.fori_loop(0, HALF_V // L, zero_body, (), unroll=8)

    pltpu.sync_copy(ids_hbm.at[pl.ds(batch * SEQ, SEQ)], ids_v)

    @pl.when(batch < B - NB_SC)
    def _():
        pltpu.sync_copy(tw_tc_hbm.at[pl.ds(batch * SEQ, SEQ)], tw_v)

    @pl.when(batch >= B - NB_SC)
    def _():
        pltpu.sync_copy(tw_sc_hbm.at[pl.ds((batch - (B - NB_SC)) * SEQ, SEQ)], tw_v)

    def tok_body(j, _):
        idx = ids_v[pl.ds(j * L, L)] - lo
        w = tw_v[pl.ds(j * L, L)]
        in_r = (idx >= 0) & (idx < HALF_V)
        idx_c = jnp.clip(idx, 0, HALF_V - 1)
        cur = plsc.load_gather(table_v, [idx_c])

        def cond(cur):
            return jnp.any(in_r & (w > cur))

        def body(cur):
            plsc.store_scatter(table_v, [idx_c], w, mask=in_r & (w > cur))
            return plsc.load_gather(table_v, [idx_c])

        lax.while_loop(cond, body, cur)
        return ()

    lax.fori_loop(0, SEQ // L, tok_body, ())

    pltpu.sync_copy(table_v, out_hbm.at[pl.ds(batch * V + lo, HALF_V)])


_sc_scatter = functools.partial(
    pl.kernel,
    out_type=jax.ShapeDtypeStruct((B * V,), jnp.float32),
    mesh=plsc.VectorSubcoreMesh(core_axis_name="c", subcore_axis_name="s"),
    compiler_params=pltpu.CompilerParams(needs_layout_passes=False),
    scratch_types=[
        pltpu.VMEM((HALF_V,), jnp.float32),
        pltpu.VMEM((SEQ,), jnp.int32),
        pltpu.VMEM((SEQ,), jnp.float32),
    ],
)(_sc_scatter_body)


@jax.jit
def kernel(hidden_states, extend_seq_lens, input_ids, W, b):
    del extend_seq_lens
    ids = input_ids.astype(jnp.int32)
    w_flat = W.reshape(H)
    b_vec = jnp.broadcast_to(b.astype(jnp.float32), (L,))
    tw_sc = _sc_matvec(hidden_states, w_flat, b_vec)
    tw_tc = _token_weights_tc(hidden_states, W, b).reshape(TC_ROWS)
    flat = _sc_scatter(ids, tw_tc, tw_sc)
    return flat.reshape(B, V)


# speedup vs baseline: 1.0187x; 1.0187x over previous
import functools

import jax
import jax.numpy as jnp
from jax import lax
from jax.experimental import pallas as pl
from jax.experimental.pallas import tpu as pltpu
from jax.experimental.pallas import tpu_sc as plsc

B = 16
SEQ = 2048
TOTAL = B * SEQ
H = 1024
V = 100000
L = 16
HC = H // L

ROWS_PER_BLOCK = 2048

NB_SC = 7
TC_ROWS = (B - NB_SC) * SEQ
SC_ROWS = NB_SC * SEQ
SC_BASE_ROW = (B - NB_SC) * SEQ
NTILE = 32
ROWS_PER_TILE = SC_ROWS // NTILE
RCHUNK = 16
NCH = ROWS_PER_TILE // RCHUNK
NPAIR = NCH // 2


def _tw_body(hs_ref, w_ref, b_ref, out_ref):
    acc = jnp.dot(hs_ref[...], w_ref[...], preferred_element_type=jnp.float32)
    out_ref[...] = jnp.maximum(acc + b_ref[0, 0], 0.0)


def _token_weights_tc(hidden_states, W, b):
    return pl.pallas_call(
        _tw_body,
        grid=(TC_ROWS // ROWS_PER_BLOCK,),
        in_specs=[
            pl.BlockSpec((ROWS_PER_BLOCK, H), lambda i: (i, 0)),
            pl.BlockSpec((H, 1), lambda i: (0, 0)),
            pl.BlockSpec(memory_space=pltpu.SMEM),
        ],
        out_specs=pl.BlockSpec((ROWS_PER_BLOCK, 1), lambda i: (i, 0)),
        out_shape=jax.ShapeDtypeStruct((TC_ROWS, 1), jnp.float32),
    )(hidden_states, W, b.reshape(1, 1))


def _sc_matvec_body(hs_hbm, w_hbm, b_hbm, tw_hbm, buf0, buf1, w_v, b_v, tw_v,
                    sem0, sem1):
    cid = lax.axis_index("c")
    sid = lax.axis_index("s")
    tile = sid * 2 + cid
    row0 = SC_BASE_ROW + tile * ROWS_PER_TILE

    pltpu.sync_copy(w_hbm, w_v)
    pltpu.sync_copy(b_hbm, b_v)
    bias = b_v[...]
    lanes = lax.broadcasted_iota(jnp.int32, (L,), 0)

    def start(g, buf, sem):
        pltpu.async_copy(hs_hbm.at[pl.ds(row0 + g * RCHUNK, RCHUNK), :], buf, sem)

    def wait(g, buf, sem):
        pltpu.make_async_copy(
            hs_hbm.at[pl.ds(row0 + g * RCHUNK, RCHUNK), :], buf, sem
        ).wait()

    def compute(buf, g):
        def c_body(c, accs):
            wv = w_v[pl.ds(c * L, L)]
            return tuple(
                accs[r] + buf[r, pl.ds(c * L, L)] * wv for r in range(RCHUNK)
            )

        accs = lax.fori_loop(
            0, HC, c_body,
            tuple(jnp.zeros((L,), jnp.float32) for _ in range(RCHUNK)),
        )
        res = jnp.zeros((L,), jnp.float32)
        for r in range(RCHUNK):
            res = jnp.where(lanes == r, jnp.sum(accs[r]), res)
        tw_v[pl.ds(g * RCHUNK, RCHUNK)] = jnp.maximum(res + bias, 0.0)

    start(0, buf0, sem0)
    start(1, buf1, sem1)

    def pair_body(p, _):
        g0 = p * 2
        wait(g0, buf0, sem0)
        compute(buf0, g0)

        @pl.when(p < NPAIR - 1)
        def _():
            start(g0 + 2, buf0, sem0)

        g1 = g0 + 1
        wait(g1, buf1, sem1)
        compute(buf1, g1)

        @pl.when(p < NPAIR - 1)
        def _():
            start(g1 + 2, buf1, sem1)

        return ()

    lax.fori_loop(0, NPAIR, pair_body, ())

    pltpu.sync_copy(tw_v, tw_hbm.at[pl.ds(tile * ROWS_PER_TILE, ROWS_PER_TILE)])


_sc_matvec = functools.partial(
    pl.kernel,
    out_type=jax.ShapeDtypeStruct((SC_ROWS,), jnp.float32),
    mesh=plsc.VectorSubcoreMesh(core_axis_name="c", subcore_axis_name="s"),
    compiler_params=pltpu.CompilerParams(needs_layout_passes=False),
    scratch_types=[
        pltpu.VMEM((RCHUNK, H), jnp.float32),
        pltpu.VMEM((RCHUNK, H), jnp.float32),
        pltpu.VMEM((H,), jnp.float32),
        pltpu.VMEM((L,), jnp.float32),
        pltpu.VMEM((ROWS_PER_TILE,), jnp.float32),
        pltpu.SemaphoreType.DMA,
        pltpu.SemaphoreType.DMA,
    ],
)(_sc_matvec_body)


HALF_V = V // 2


def _sc_scatter_body(ids_hbm, tw_tc_hbm, tw_sc_hbm, out_hbm, table_v, ids_v, tw_v):
    cid = lax.axis_index("c")
    sid = lax.axis_index("s")
    wid = sid * 2 + cid
    batch = wid // 2
    lo = (wid % 2) * HALF_V

    zeros = jnp.zeros((L,), jnp.float32)

    def zero_body(j, _):
        table_v[pl.ds(j * L, L)] = zeros
        return ()

    lax.fori_loop(0, HALF_V // L, zero_body, (), unroll=8)

    pltpu.sync_copy(ids_hbm.at[pl.ds(batch * SEQ, SEQ)], ids_v)

    @pl.when(batch < B - NB_SC)
    def _():
        pltpu.sync_copy(tw_tc_hbm.at[pl.ds(batch * SEQ, SEQ)], tw_v)

    @pl.when(batch >= B - NB_SC)
    def _():
        pltpu.sync_copy(tw_sc_hbm.at[pl.ds((batch - (B - NB_SC)) * SEQ, SEQ)], tw_v)

    def tok_body(j, _):
        idx = ids_v[pl.ds(j * L, L)] - lo
        w = tw_v[pl.ds(j * L, L)]
        in_r = (idx >= 0) & (idx < HALF_V)
        idx_c = jnp.clip(idx, 0, HALF_V - 1)
        cur = plsc.load_gather(table_v, [idx_c])

        def cond(cur):
            return jnp.any(in_r & (w > cur))

        def body(cur):
            plsc.store_scatter(table_v, [idx_c], w, mask=in_r & (w > cur))
            return plsc.load_gather(table_v, [idx_c])

        lax.while_loop(cond, body, cur)
        return ()

    lax.fori_loop(0, SEQ // L, tok_body, ())

    pltpu.sync_copy(table_v, out_hbm.at[pl.ds(batch * V + lo, HALF_V)])


_sc_scatter = functools.partial(
    pl.kernel,
    out_type=jax.ShapeDtypeStruct((B * V,), jnp.float32),
    mesh=plsc.VectorSubcoreMesh(core_axis_name="c", subcore_axis_name="s"),
    compiler_params=pltpu.CompilerParams(needs_layout_passes=False),
    scratch_types=[
        pltpu.VMEM((HALF_V,), jnp.float32),
        pltpu.VMEM((SEQ,), jnp.int32),
        pltpu.VMEM((SEQ,), jnp.float32),
    ],
)(_sc_scatter_body)


@jax.jit
def kernel(hidden_states, extend_seq_lens, input_ids, W, b):
    del extend_seq_lens
    ids = input_ids.astype(jnp.int32)
    w_flat = W.reshape(H)
    b_vec = jnp.broadcast_to(b.astype(jnp.float32), (L,))
    tw_sc = _sc_matvec(hidden_states, w_flat, b_vec)
    tw_tc = _token_weights_tc(hidden_states, W, b).reshape(TC_ROWS)
    flat = _sc_scatter(ids, tw_tc, tw_sc)
    return flat.reshape(B, V)


# speedup vs baseline: 1.0282x; 1.0093x over previous
import functools

import jax
import jax.numpy as jnp
from jax import lax
from jax.experimental import pallas as pl
from jax.experimental.pallas import tpu as pltpu
from jax.experimental.pallas import tpu_sc as plsc

B = 16
SEQ = 2048
TOTAL = B * SEQ
H = 1024
V = 100000
L = 16
HC = H // L

ROWS_PER_BLOCK = 2048

NB_SC = 7
TC_ROWS = (B - NB_SC) * SEQ
SC_ROWS = NB_SC * SEQ
SC_BASE_ROW = (B - NB_SC) * SEQ
NTILE = 32
ROWS_PER_TILE = SC_ROWS // NTILE
RCHUNK = 16
NCH = ROWS_PER_TILE // RCHUNK
NPAIR = NCH // 2


def _tw_body(hs_ref, w_ref, b_ref, out_ref):
    acc = jnp.dot(hs_ref[...], w_ref[...], preferred_element_type=jnp.float32)
    out_ref[...] = jnp.maximum(acc + b_ref[0, 0], 0.0)


def _token_weights_tc(hidden_states, W, b):
    return pl.pallas_call(
        _tw_body,
        grid=(TC_ROWS // ROWS_PER_BLOCK,),
        in_specs=[
            pl.BlockSpec((ROWS_PER_BLOCK, H), lambda i: (i, 0)),
            pl.BlockSpec((H, 1), lambda i: (0, 0)),
            pl.BlockSpec(memory_space=pltpu.SMEM),
        ],
        out_specs=pl.BlockSpec((ROWS_PER_BLOCK, 1), lambda i: (i, 0)),
        out_shape=jax.ShapeDtypeStruct((TC_ROWS, 1), jnp.float32),
    )(hidden_states, W, b.reshape(1, 1))


def _sc_matvec_body(hs_hbm, w_hbm, b_hbm, tw_hbm, buf0, buf1, w_v, b_v, tw_v,
                    sem0, sem1):
    cid = lax.axis_index("c")
    sid = lax.axis_index("s")
    tile = sid * 2 + cid
    row0 = SC_BASE_ROW + tile * ROWS_PER_TILE

    pltpu.sync_copy(w_hbm, w_v)
    pltpu.sync_copy(b_hbm, b_v)
    bias = b_v[...]
    lanes = lax.broadcasted_iota(jnp.int32, (L,), 0)

    def start(g, buf, sem):
        pltpu.async_copy(hs_hbm.at[pl.ds(row0 + g * RCHUNK, RCHUNK), :], buf, sem)

    def wait(g, buf, sem):
        pltpu.make_async_copy(
            hs_hbm.at[pl.ds(row0 + g * RCHUNK, RCHUNK), :], buf, sem
        ).wait()

    def compute(buf, g):
        def c_body(c, accs):
            wv = w_v[pl.ds(c * L, L)]
            return tuple(
                accs[r] + buf[r, pl.ds(c * L, L)] * wv for r in range(RCHUNK)
            )

        accs = lax.fori_loop(
            0, HC, c_body,
            tuple(jnp.zeros((L,), jnp.float32) for _ in range(RCHUNK)),
            unroll=4,
        )
        res = jnp.zeros((L,), jnp.float32)
        for r in range(RCHUNK):
            res = jnp.where(lanes == r, jnp.sum(accs[r]), res)
        tw_v[pl.ds(g * RCHUNK, RCHUNK)] = jnp.maximum(res + bias, 0.0)

    start(0, buf0, sem0)
    start(1, buf1, sem1)

    def pair_body(p, _):
        g0 = p * 2
        wait(g0, buf0, sem0)
        compute(buf0, g0)

        @pl.when(p < NPAIR - 1)
        def _():
            start(g0 + 2, buf0, sem0)

        g1 = g0 + 1
        wait(g1, buf1, sem1)
        compute(buf1, g1)

        @pl.when(p < NPAIR - 1)
        def _():
            start(g1 + 2, buf1, sem1)

        return ()

    lax.fori_loop(0, NPAIR, pair_body, ())

    pltpu.sync_copy(tw_v, tw_hbm.at[pl.ds(tile * ROWS_PER_TILE, ROWS_PER_TILE)])


_sc_matvec = functools.partial(
    pl.kernel,
    out_type=jax.ShapeDtypeStruct((SC_ROWS,), jnp.float32),
    mesh=plsc.VectorSubcoreMesh(core_axis_name="c", subcore_axis_name="s"),
    compiler_params=pltpu.CompilerParams(needs_layout_passes=False),
    scratch_types=[
        pltpu.VMEM((RCHUNK, H), jnp.float32),
        pltpu.VMEM((RCHUNK, H), jnp.float32),
        pltpu.VMEM((H,), jnp.float32),
        pltpu.VMEM((L,), jnp.float32),
        pltpu.VMEM((ROWS_PER_TILE,), jnp.float32),
        pltpu.SemaphoreType.DMA,
        pltpu.SemaphoreType.DMA,
    ],
)(_sc_matvec_body)


HALF_V = V // 2


def _sc_scatter_body(ids_hbm, tw_tc_hbm, tw_sc_hbm, out_hbm, table_v, ids_v, tw_v):
    cid = lax.axis_index("c")
    sid = lax.axis_index("s")
    wid = sid * 2 + cid
    batch = wid // 2
    lo = (wid % 2) * HALF_V

    zeros = jnp.zeros((L,), jnp.float32)

    def zero_body(j, _):
        table_v[pl.ds(j * L, L)] = zeros
        return ()

    lax.fori_loop(0, HALF_V // L, zero_body, (), unroll=8)

    pltpu.sync_copy(ids_hbm.at[pl.ds(batch * SEQ, SEQ)], ids_v)

    @pl.when(batch < B - NB_SC)
    def _():
        pltpu.sync_copy(tw_tc_hbm.at[pl.ds(batch * SEQ, SEQ)], tw_v)

    @pl.when(batch >= B - NB_SC)
    def _():
        pltpu.sync_copy(tw_sc_hbm.at[pl.ds((batch - (B - NB_SC)) * SEQ, SEQ)], tw_v)

    def tok_body(j, _):
        idx = ids_v[pl.ds(j * L, L)] - lo
        w = tw_v[pl.ds(j * L, L)]
        in_r = (idx >= 0) & (idx < HALF_V)
        idx_c = jnp.clip(idx, 0, HALF_V - 1)
        cur = plsc.load_gather(table_v, [idx_c])

        def cond(cur):
            return jnp.any(in_r & (w > cur))

        def body(cur):
            plsc.store_scatter(table_v, [idx_c], w, mask=in_r & (w > cur))
            return plsc.load_gather(table_v, [idx_c])

        lax.while_loop(cond, body, cur)
        return ()

    lax.fori_loop(0, SEQ // L, tok_body, ())

    pltpu.sync_copy(table_v, out_hbm.at[pl.ds(batch * V + lo, HALF_V)])


_sc_scatter = functools.partial(
    pl.kernel,
    out_type=jax.ShapeDtypeStruct((B * V,), jnp.float32),
    mesh=plsc.VectorSubcoreMesh(core_axis_name="c", subcore_axis_name="s"),
    compiler_params=pltpu.CompilerParams(needs_layout_passes=False),
    scratch_types=[
        pltpu.VMEM((HALF_V,), jnp.float32),
        pltpu.VMEM((SEQ,), jnp.int32),
        pltpu.VMEM((SEQ,), jnp.float32),
    ],
)(_sc_scatter_body)


@jax.jit
def kernel(hidden_states, extend_seq_lens, input_ids, W, b):
    del extend_seq_lens
    ids = input_ids.astype(jnp.int32)
    w_flat = W.reshape(H)
    b_vec = jnp.broadcast_to(b.astype(jnp.float32), (L,))
    tw_sc = _sc_matvec(hidden_states, w_flat, b_vec)
    tw_tc = _token_weights_tc(hidden_states, W, b).reshape(TC_ROWS)
    flat = _sc_scatter(ids, tw_tc, tw_sc)
    return flat.reshape(B, V)
